# baseline (speedup 1.0000x reference)
import functools

import jax
import jax.numpy as jnp
from jax import lax
from jax.experimental import pallas as pl
from jax.experimental.pallas import tpu as pltpu
from jax.experimental.pallas import tpu_sc as plsc

VOCAB = 1000000
SEQ = 200
DIM = 64
BATCH = 4096

NC = 2
NS = 16
NW = NC * NS
LANES = 16

BPW = BATCH // NW
NBUF = 2
SCALE = 8.0

_mesh = plsc.VectorSubcoreMesh(core_axis_name="c", subcore_axis_name="s")


@functools.partial(
    pl.kernel,
    out_type=jax.ShapeDtypeStruct((SEQ, DIM // 8, NW, 8, 128), jnp.float32),
    mesh=_mesh,
    compiler_params=pltpu.CompilerParams(use_tc_tiling_on_sc=True, needs_layout_passes=False),
    scratch_types=[
        pltpu.VMEM((SEQ, BPW), jnp.int32),
        pltpu.VMEM((NBUF, BPW), jnp.int32),
        pltpu.VMEM((NBUF, BPW, 128), jnp.float32),
        pltpu.VMEM((NBUF, DIM // 8, 8, 128), jnp.float32),
        pltpu.VMEM((SEQ, DIM), jnp.float32),
        pltpu.SemaphoreType.DMA((NBUF,)),
        pltpu.SemaphoreType.DMA((NBUF,)),
    ],
)
def _embed(tok2_hbm, idxT_hbm, pos_hbm, out_hbm,
           idxT_v, idx2_v, rows_v, out_v, pos_v, gsem, osem):
    wid = lax.axis_index("s") * NC + lax.axis_index("c")
    pltpu.sync_copy(idxT_hbm.at[:, pl.ds(wid * BPW, BPW)], idxT_v)
    pltpu.sync_copy(pos_hbm, pos_v)

    def prep(s, b):
        for g in range(BPW // LANES):
            sl = pl.ds(g * LANES, LANES)
            idx2_v[b, sl] = lax.shift_right_logical(idxT_v[s, sl], 1)

    def fire_gather(b):
        pltpu.async_copy(tok2_hbm.at[idx2_v.at[b]], rows_v.at[b], gsem.at[b])

    def wait_gather(b):
        pltpu.make_async_copy(
            tok2_hbm.at[idx2_v.at[b]], rows_v.at[b], gsem.at[b]
        ).wait()

    def fire_store(s, b):
        pltpu.async_copy(out_v.at[b], out_hbm.at[s, :, wid], osem.at[b])

    def wait_store(s, b):
        pltpu.make_async_copy(
            out_v.at[b], out_hbm.at[s, :, wid], osem.at[b]
        ).wait()

    def compute(s, b):
        rows = rows_v.at[b]
        rowcols = []
        for g in range(BPW // LANES):
            sl = pl.ds(g * LANES, LANES)
            tv = idxT_v[s, sl]
            par = lax.bitwise_and(tv, 1)
            row = lax.iota(jnp.int32, 16) + (g * LANES)
            colb = par * DIM
            rowcols.append((row, colb))
        pvecs = [
            pos_v[s, pl.ds(k * LANES, LANES)] for k in range(DIM // LANES)
        ]
        for d in range(DIM):
            p = jnp.broadcast_to(pvecs[d // LANES][d % LANES], (16,))
            for g in range(BPW // LANES):
                row, colb = rowcols[g]
                v = plsc.load_gather(rows, [row, colb + d])
                out_v[b, d // 8, d % 8, pl.ds(g * LANES, LANES)] = (
                    v * SCALE + p
                )

    prep(0, 0)
    fire_gather(0)

    def pair_body(pr, carry):
        s0 = 2 * pr

        prep(s0 + 1, 1)
        fire_gather(1)
        wait_gather(0)

        @pl.when(pr >= 1)
        def _():
            wait_store(s0 - 2, 0)

        compute(s0, 0)
        fire_store(s0, 0)

        @pl.when(pr + 1 < SEQ // 2)
        def _():
            prep(s0 + 2, 0)
            fire_gather(0)

        wait_gather(1)

        @pl.when(pr >= 1)
        def _():
            wait_store(s0 - 1, 1)

        compute(s0 + 1, 1)
        fire_store(s0 + 1, 1)
        return carry

    lax.fori_loop(0, SEQ // 2, pair_body, 0)
    wait_store(SEQ - 2, 0)
    wait_store(SEQ - 1, 1)


def kernel(inputs, token_table, position_table):
    tok2 = token_table.reshape(VOCAB // 2, 128)
    idxT = inputs.T
    out5 = _embed(tok2, idxT, position_table)
    return out5.transpose(2, 4, 0, 1, 3).reshape(BATCH, SEQ, DIM)


# speedup vs baseline: 1.7599x; 1.7599x over previous
import functools

import jax
import jax.numpy as jnp
from jax import lax
from jax.experimental import pallas as pl
from jax.experimental.pallas import tpu as pltpu
from jax.experimental.pallas import tpu_sc as plsc

VOCAB = 1000000
SEQ = 200
DIM = 64
BATCH = 4096

NC = 2
NS = 16
NW = NC * NS
LANES = 16

BPW = BATCH // NW
NBUF = 2
SCALE = 8.0

_mesh = plsc.VectorSubcoreMesh(core_axis_name="c", subcore_axis_name="s")


@functools.partial(
    pl.kernel,
    out_type=jax.ShapeDtypeStruct((SEQ, DIM // 8, NW, 8, 128), jnp.float32),
    mesh=_mesh,
    compiler_params=pltpu.CompilerParams(use_tc_tiling_on_sc=True, needs_layout_passes=False),
    scratch_types=[
        pltpu.VMEM((SEQ, BPW), jnp.int32),
        pltpu.VMEM((NBUF, BPW), jnp.int32),
        pltpu.VMEM((NBUF, BPW, 128), jnp.float32),
        pltpu.VMEM((NBUF, DIM // 8, 8, 128), jnp.float32),
        pltpu.VMEM((SEQ, DIM), jnp.float32),
        pltpu.SemaphoreType.DMA((NBUF,)),
        pltpu.SemaphoreType.DMA((NBUF,)),
    ],
)
def _embed(tok2_hbm, idxT_hbm, pos_hbm, out_hbm,
           idxT_v, idx2_v, rows_v, out_v, pos_v, gsem, osem):
    wid = lax.axis_index("s") * NC + lax.axis_index("c")
    pltpu.sync_copy(idxT_hbm.at[:, pl.ds(wid * BPW, BPW)], idxT_v)
    pltpu.sync_copy(pos_hbm, pos_v)

    def prep(s, b):
        for g in range(BPW // LANES):
            sl = pl.ds(g * LANES, LANES)
            idx2_v[b, sl] = lax.shift_right_logical(idxT_v[s, sl], 1)

    def fire_gather(b):
        pltpu.async_copy(tok2_hbm.at[idx2_v.at[b]], rows_v.at[b], gsem.at[b])

    def wait_gather(b):
        pltpu.make_async_copy(
            tok2_hbm.at[idx2_v.at[b]], rows_v.at[b], gsem.at[b]
        ).wait()

    def fire_store(s, b):
        pltpu.async_copy(out_v.at[b], out_hbm.at[s, :, wid], osem.at[b])

    def wait_store(s, b):
        pltpu.make_async_copy(
            out_v.at[b], out_hbm.at[s, :, wid], osem.at[b]
        ).wait()

    iota = lax.iota(jnp.int32, 16)

    def compute(s, b):
        rows = rows_v.at[b]
        out3 = out_v.at[b]
        svec = jnp.broadcast_to(s, (16,))
        rowv, colp = [], []
        for g in range(BPW // LANES):
            tv = idxT_v[s, pl.ds(g * LANES, LANES)]
            rowv.append(iota + (g * LANES))
            colp.append(lax.bitwise_and(tv, 1) * DIM)

        def k_body(k, carry):
            crowv, ccolp = carry
            perm = lax.rem(iota + k, 16)
            phi = lax.shift_right_logical(perm, 3)
            plo = lax.bitwise_and(perm, 7)
            for db in range(DIM // LANES):
                dcol = perm + (db * LANES)
                dhi = phi + (db * 2)
                p = plsc.load_gather(pos_v, [svec, dcol])
                for g in range(BPW // LANES):
                    v = plsc.load_gather(rows, [crowv[g], ccolp[g] + dcol])
                    plsc.store_scatter(out3, [dhi, plo, crowv[g]],
                                       v * SCALE + p)
            return carry

        lax.fori_loop(0, LANES, k_body, (tuple(rowv), tuple(colp)))

    prep(0, 0)
    fire_gather(0)

    def pair_body(pr, carry):
        s0 = 2 * pr

        prep(s0 + 1, 1)
        fire_gather(1)
        wait_gather(0)

        @pl.when(pr >= 1)
        def _():
            wait_store(s0 - 2, 0)

        compute(s0, 0)
        fire_store(s0, 0)

        @pl.when(pr + 1 < SEQ // 2)
        def _():
            prep(s0 + 2, 0)
            fire_gather(0)

        wait_gather(1)

        @pl.when(pr >= 1)
        def _():
            wait_store(s0 - 1, 1)

        compute(s0 + 1, 1)
        fire_store(s0 + 1, 1)
        return carry

    lax.fori_loop(0, SEQ // 2, pair_body, 0)
    wait_store(SEQ - 2, 0)
    wait_store(SEQ - 1, 1)


def kernel(inputs, token_table, position_table):
    tok2 = token_table.reshape(VOCAB // 2, 128)
    idxT = inputs.T
    out5 = _embed(tok2, idxT, position_table)
    return out5.transpose(2, 4, 0, 1, 3).reshape(BATCH, SEQ, DIM)
